# Initial kernel scaffold; baseline (speedup 1.0000x reference)
import jax
import jax.numpy as jnp
from jax.experimental import pallas as pl


def kernel(x, expert_weights, expert_indices, top_k, w1, s1, b1, w2, s2, b2):
    raise NotImplementedError("write your pallas kernel here")


# speedup vs baseline: 2.3568x; 2.3568x over previous
import jax
import jax.numpy as jnp
from jax.experimental import pallas as pl
from jax.experimental.pallas import tpu as pltpu


def _moe_tile_body(te_ref, x_ref, rw_ref, w1_ref, s1_ref, b1_ref,
                   w2_ref, s2_ref, b2_ref, o_ref):
    ng1, two_i = s1_ref.shape[1], s1_ref.shape[2]
    h = x_ref.shape[1]
    g1 = h // ng1
    ii = w2_ref.shape[1]
    ng2 = s2_ref.shape[1]
    g2 = ii // ng2

    x = x_ref[...]
    q1 = w1_ref[0].astype(jnp.float32)
    s1f = jnp.broadcast_to(s1_ref[0][:, None, :], (ng1, g1, two_i)).reshape(h, two_i)
    b1f = jnp.broadcast_to(b1_ref[0][:, None, :], (ng1, g1, two_i)).reshape(h, two_i)
    w1f = q1 * s1f + b1f
    y = jnp.dot(x, w1f, preferred_element_type=jnp.float32)
    half = two_i // 2
    gate = y[:, :half]
    up = y[:, half:]
    act = gate * jax.nn.sigmoid(gate) * up

    q2 = w2_ref[0].astype(jnp.float32)
    s2f = jnp.broadcast_to(s2_ref[0][:, None, :], (ng2, g2, h)).reshape(ii, h)
    b2f = jnp.broadcast_to(b2_ref[0][:, None, :], (ng2, g2, h)).reshape(ii, h)
    w2f = q2 * s2f + b2f
    z = jnp.dot(act, w2f, preferred_element_type=jnp.float32)
    o_ref[...] = z * rw_ref[0, 0][:, None]


def kernel(x, expert_weights, expert_indices, top_k, w1, s1, b1, w2, s2, b2):
    p, h = x.shape
    e, two_i, _ = w1.shape
    i = w2.shape[2]
    tk = expert_indices.shape[-1]
    t = p * tk
    B = 256
    nt = t // B + e

    flat = expert_indices.reshape(-1).astype(jnp.int32)
    order = jnp.argsort(flat).astype(jnp.int32)
    sorted_e = flat[order]
    counts = jnp.bincount(flat, length=e).astype(jnp.int32)
    tiles_per = (counts + B - 1) // B
    tile_start = jnp.cumsum(tiles_per) - tiles_per
    offs = jnp.cumsum(counts) - counts
    jpos = jnp.arange(t, dtype=jnp.int32)
    dest = tile_start[sorted_e] * B + (jpos - offs[sorted_e])
    src = jnp.zeros((nt * B,), jnp.int32).at[dest].set(order)
    rw_flat = expert_weights.reshape(-1).astype(jnp.float32)
    rw_pad = jnp.zeros((nt * B,), jnp.float32).at[dest].set(rw_flat[order])
    tile_expert = jnp.minimum(
        jnp.searchsorted(jnp.cumsum(tiles_per), jnp.arange(nt), side="right"),
        e - 1).astype(jnp.int32)

    x_pad = jnp.take(x, src // tk, axis=0)
    rw3 = rw_pad.reshape(nt, 1, B)

    w1t = w1.astype(jnp.int8).transpose(0, 2, 1)
    s1t = s1.transpose(0, 2, 1)
    b1t = b1.transpose(0, 2, 1)
    w2t = w2.astype(jnp.int8).transpose(0, 2, 1)
    s2t = s2.transpose(0, 2, 1)
    b2t = b2.transpose(0, 2, 1)
    ng1 = s1t.shape[1]
    ng2 = s2t.shape[1]

    grid_spec = pltpu.PrefetchScalarGridSpec(
        num_scalar_prefetch=1,
        grid=(nt,),
        in_specs=[
            pl.BlockSpec((B, h), lambda ti, te: (ti, 0)),
            pl.BlockSpec((1, 1, B), lambda ti, te: (ti, 0, 0)),
            pl.BlockSpec((1, h, two_i), lambda ti, te: (te[ti], 0, 0)),
            pl.BlockSpec((1, ng1, two_i), lambda ti, te: (te[ti], 0, 0)),
            pl.BlockSpec((1, ng1, two_i), lambda ti, te: (te[ti], 0, 0)),
            pl.BlockSpec((1, i, h), lambda ti, te: (te[ti], 0, 0)),
            pl.BlockSpec((1, ng2, h), lambda ti, te: (te[ti], 0, 0)),
            pl.BlockSpec((1, ng2, h), lambda ti, te: (te[ti], 0, 0)),
        ],
        out_specs=pl.BlockSpec((B, h), lambda ti, te: (ti, 0)),
    )
    z_pad = pl.pallas_call(
        _moe_tile_body,
        grid_spec=grid_spec,
        out_shape=jax.ShapeDtypeStruct((nt * B, h), jnp.float32),
    )(tile_expert, x_pad, rw3, w1t, s1t, b1t, w2t, s2t, b2t)

    dest_flat = jnp.zeros((t,), jnp.int32).at[order].set(dest)
    return jnp.take(z_pad, dest_flat, axis=0).reshape(p, tk, h).sum(axis=1)


# speedup vs baseline: 2.5441x; 1.0795x over previous
import jax
import jax.numpy as jnp
from jax.experimental import pallas as pl
from jax.experimental.pallas import tpu as pltpu


def _moe_tile_body(te_ref, x_ref, rw_ref, w1_ref, s1_ref, b1_ref,
                   w2_ref, s2_ref, b2_ref, o_ref, w1f_ref, w2f_ref):
    ng1, two_i = s1_ref.shape[1], s1_ref.shape[2]
    h = x_ref.shape[1]
    g1 = h // ng1
    ii = w2_ref.shape[1]
    ng2 = s2_ref.shape[1]
    g2 = ii // ng2

    t = pl.program_id(0)
    te_now = te_ref[t]
    te_prev = te_ref[jnp.maximum(t - 1, 0)]

    @pl.when((t == 0) | (te_now != te_prev))
    def _dequant():
        q1 = w1_ref[0].astype(jnp.bfloat16)
        s1f = jnp.broadcast_to(s1_ref[0][:, None, :], (ng1, g1, two_i)).reshape(h, two_i)
        b1f = jnp.broadcast_to(b1_ref[0][:, None, :], (ng1, g1, two_i)).reshape(h, two_i)
        w1f_ref[...] = q1 * s1f + b1f
        q2 = w2_ref[0].astype(jnp.bfloat16)
        s2f = jnp.broadcast_to(s2_ref[0][:, None, :], (ng2, g2, h)).reshape(ii, h)
        b2f = jnp.broadcast_to(b2_ref[0][:, None, :], (ng2, g2, h)).reshape(ii, h)
        w2f_ref[...] = q2 * s2f + b2f

    x = x_ref[...]
    y = jnp.dot(x, w1f_ref[...], preferred_element_type=jnp.float32)
    half = two_i // 2
    gate = y[:, :half]
    up = y[:, half:]
    act = gate * jax.nn.sigmoid(gate) * up
    z = jnp.dot(act.astype(jnp.bfloat16), w2f_ref[...],
                preferred_element_type=jnp.float32)
    o_ref[...] = z * rw_ref[0, 0][:, None]


def kernel(x, expert_weights, expert_indices, top_k, w1, s1, b1, w2, s2, b2):
    p, h = x.shape
    e, two_i, _ = w1.shape
    i = w2.shape[2]
    tk = expert_indices.shape[-1]
    t = p * tk
    B = 128
    nt = (t + e * (B - 1)) // B

    flat = expert_indices.reshape(-1).astype(jnp.int32)
    order = jnp.argsort(flat).astype(jnp.int32)
    sorted_e = flat[order]
    counts = jnp.bincount(flat, length=e).astype(jnp.int32)
    tiles_per = (counts + B - 1) // B
    tile_start = jnp.cumsum(tiles_per) - tiles_per
    offs = jnp.cumsum(counts) - counts
    jpos = jnp.arange(t, dtype=jnp.int32)
    dest = tile_start[sorted_e] * B + (jpos - offs[sorted_e])
    src = jnp.zeros((nt * B,), jnp.int32).at[dest].set(order)
    rw_flat = expert_weights.reshape(-1).astype(jnp.float32)
    rw_pad = jnp.zeros((nt * B,), jnp.float32).at[dest].set(rw_flat[order])
    tile_expert = jnp.minimum(
        jnp.searchsorted(jnp.cumsum(tiles_per), jnp.arange(nt), side="right"),
        e - 1).astype(jnp.int32)

    x_pad = jnp.take(x.astype(jnp.bfloat16), src // tk, axis=0)
    rw3 = rw_pad.reshape(nt, 1, B)

    w1t = w1.astype(jnp.int8).transpose(0, 2, 1)
    s1t = s1.astype(jnp.bfloat16).transpose(0, 2, 1)
    b1t = b1.astype(jnp.bfloat16).transpose(0, 2, 1)
    w2t = w2.astype(jnp.int8).transpose(0, 2, 1)
    s2t = s2.astype(jnp.bfloat16).transpose(0, 2, 1)
    b2t = b2.astype(jnp.bfloat16).transpose(0, 2, 1)
    ng1 = s1t.shape[1]
    ng2 = s2t.shape[1]

    grid_spec = pltpu.PrefetchScalarGridSpec(
        num_scalar_prefetch=1,
        grid=(nt,),
        in_specs=[
            pl.BlockSpec((B, h), lambda ti, te: (ti, 0)),
            pl.BlockSpec((1, 1, B), lambda ti, te: (ti, 0, 0)),
            pl.BlockSpec((1, h, two_i), lambda ti, te: (te[ti], 0, 0)),
            pl.BlockSpec((1, ng1, two_i), lambda ti, te: (te[ti], 0, 0)),
            pl.BlockSpec((1, ng1, two_i), lambda ti, te: (te[ti], 0, 0)),
            pl.BlockSpec((1, i, h), lambda ti, te: (te[ti], 0, 0)),
            pl.BlockSpec((1, ng2, h), lambda ti, te: (te[ti], 0, 0)),
            pl.BlockSpec((1, ng2, h), lambda ti, te: (te[ti], 0, 0)),
        ],
        out_specs=pl.BlockSpec((B, h), lambda ti, te: (ti, 0)),
        scratch_shapes=[
            pltpu.VMEM((h, two_i), jnp.bfloat16),
            pltpu.VMEM((i, h), jnp.bfloat16),
        ],
    )
    z_pad = pl.pallas_call(
        _moe_tile_body,
        grid_spec=grid_spec,
        out_shape=jax.ShapeDtypeStruct((nt * B, h), jnp.float32),
    )(tile_expert, x_pad, rw3, w1t, s1t, b1t, w2t, s2t, b2t)

    dest_flat = jnp.zeros((t,), jnp.int32).at[order].set(dest)
    return jnp.take(z_pad, dest_flat, axis=0).reshape(p, tk, h).sum(axis=1)
